# baseline (speedup 1.0000x reference)
import functools

import jax
import jax.numpy as jnp
from jax import lax
from jax.experimental import pallas as pl
from jax.experimental.pallas import tpu as pltpu
from jax.experimental.pallas import tpu_sc as plsc

B = 16384
K = 64
NW = 32
BPW = B // NW
GROUPS = BPW // 16


def _body(uid_hbm, iid_hbm, w_hbm, u_hbm, bu_hbm, bi_hbm, gm_hbm, out_hbm,
          uid_v, iid_v, wrows, urows, bu_v, bi_v, out_v, gm_v, hbuf, sem):
    wid = lax.axis_index("s") * 2 + lax.axis_index("c")
    base = wid * BPW

    pltpu.sync_copy(uid_hbm.at[pl.ds(base, BPW)], uid_v)
    pltpu.sync_copy(iid_hbm.at[pl.ds(base, BPW)], iid_v)
    pltpu.sync_copy(gm_hbm, gm_v)

    cw = pltpu.async_copy(w_hbm.at[uid_v], wrows, sem)
    cu = pltpu.async_copy(u_hbm.at[iid_v], urows, sem)
    cbu = pltpu.async_copy(bu_hbm.at[uid_v], bu_v, sem)
    cbi = pltpu.async_copy(bi_hbm.at[iid_v], bi_v, sem)
    cw.wait()
    cu.wait()
    cbu.wait()
    cbi.wait()

    gm = gm_v[...]
    lanes16 = lax.iota(jnp.int32, 16) * 16

    def group(g, carry):
        r0 = g * 16
        for j in range(16):
            r = r0 + j
            acc = wrows[r, pl.ds(0, 16)] * urows[r, pl.ds(0, 16)]
            for c in range(1, 4):
                acc = acc + wrows[r, pl.ds(c * 16, 16)] * urows[r, pl.ds(c * 16, 16)]
            hbuf[pl.ds(j * 16, 16)] = acc
        o = plsc.load_gather(hbuf, [lanes16])
        for l in range(1, 16):
            o = o + plsc.load_gather(hbuf, [lanes16 + l])
        out_v[pl.ds(r0, 16)] = o + bu_v[pl.ds(r0, 16)] + bi_v[pl.ds(r0, 16)] + gm
        return carry

    lax.fori_loop(0, GROUPS, group, 0)
    pltpu.sync_copy(out_v, out_hbm.at[pl.ds(base, BPW)])


def kernel(user_ids, item_ids, W, U, bias_user, bias_item, global_mean):
    uid = user_ids.astype(jnp.int32)
    iid = item_ids.astype(jnp.int32)
    gm16 = jnp.broadcast_to(global_mean.astype(jnp.float32), (16,))

    run = functools.partial(
        pl.kernel,
        mesh=plsc.VectorSubcoreMesh(core_axis_name="c", subcore_axis_name="s"),
        compiler_params=pltpu.CompilerParams(
            needs_layout_passes=False, use_tc_tiling_on_sc=False),
        out_type=jax.ShapeDtypeStruct((B,), jnp.float32),
        scratch_types=[
            pltpu.VMEM((BPW,), jnp.int32),
            pltpu.VMEM((BPW,), jnp.int32),
            pltpu.VMEM((BPW, K), jnp.float32),
            pltpu.VMEM((BPW, K), jnp.float32),
            pltpu.VMEM((BPW,), jnp.float32),
            pltpu.VMEM((BPW,), jnp.float32),
            pltpu.VMEM((BPW,), jnp.float32),
            pltpu.VMEM((16,), jnp.float32),
            pltpu.VMEM((256,), jnp.float32),
            pltpu.SemaphoreType.DMA,
        ],
    )(_body)
    return run(uid, iid, W, U, bias_user, bias_item, gm16)


# speedup vs baseline: 1.5226x; 1.5226x over previous
import functools

import jax
import jax.numpy as jnp
from jax import lax
from jax.experimental import pallas as pl
from jax.experimental.pallas import tpu as pltpu
from jax.experimental.pallas import tpu_sc as plsc

B = 16384
K = 64
NW = 32
BPW = B // NW
CHUNK = 16
NCHUNK = BPW // CHUNK


def _body(uid_hbm, iid_hbm, w_hbm, u_hbm, bu_hbm, bi_hbm, gm_hbm, out_hbm,
          uid_v, iid_v, wr, ur, bu_v, bi_v, out_v, gm_v, hbuf, sem):
    wid = lax.axis_index("s") * 2 + lax.axis_index("c")
    base = wid * BPW

    pltpu.sync_copy(uid_hbm.at[pl.ds(base, BPW)], uid_v)
    pltpu.sync_copy(iid_hbm.at[pl.ds(base, BPW)], iid_v)
    pltpu.sync_copy(gm_hbm, gm_v)

    cbu = pltpu.async_copy(bu_hbm.at[uid_v], bu_v, sem)
    cbi = pltpu.async_copy(bi_hbm.at[iid_v], bi_v, sem)
    cbu.wait()
    cbi.wait()

    gm = gm_v[...]
    lanes16 = lax.iota(jnp.int32, 16) * 16

    def chunk(c, carry):
        r0 = c * CHUNK
        uvec = uid_v[pl.ds(r0, 16)]
        ivec = iid_v[pl.ds(r0, 16)]
        copies = []
        for j in range(CHUNK):
            copies.append(pltpu.async_copy(
                w_hbm.at[pl.ds(uvec[j], 1), :], wr.at[pl.ds(j, 1), :], sem))
            copies.append(pltpu.async_copy(
                u_hbm.at[pl.ds(ivec[j], 1), :], ur.at[pl.ds(j, 1), :], sem))
        for cp in copies:
            cp.wait()
        for j in range(CHUNK):
            acc = wr[j, pl.ds(0, 16)] * ur[j, pl.ds(0, 16)]
            for q in range(1, 4):
                acc = acc + (wr[j, pl.ds(q * 16, 16)]
                             * ur[j, pl.ds(q * 16, 16)])
            hbuf[pl.ds(j * 16, 16)] = acc
        o = plsc.load_gather(hbuf, [lanes16])
        for l in range(1, 16):
            o = o + plsc.load_gather(hbuf, [lanes16 + l])
        out_v[pl.ds(r0, 16)] = o + bu_v[pl.ds(r0, 16)] + bi_v[pl.ds(r0, 16)] + gm
        return carry

    lax.fori_loop(0, NCHUNK, chunk, 0)
    pltpu.sync_copy(out_v, out_hbm.at[pl.ds(base, BPW)])


def kernel(user_ids, item_ids, W, U, bias_user, bias_item, global_mean):
    uid = user_ids.astype(jnp.int32)
    iid = item_ids.astype(jnp.int32)
    gm16 = jnp.broadcast_to(global_mean.astype(jnp.float32), (16,))

    run = functools.partial(
        pl.kernel,
        mesh=plsc.VectorSubcoreMesh(core_axis_name="c", subcore_axis_name="s"),
        compiler_params=pltpu.CompilerParams(needs_layout_passes=False),
        out_type=jax.ShapeDtypeStruct((B,), jnp.float32),
        scratch_types=[
            pltpu.VMEM((BPW,), jnp.int32),
            pltpu.VMEM((BPW,), jnp.int32),
            pltpu.VMEM((CHUNK, K), jnp.float32),
            pltpu.VMEM((CHUNK, K), jnp.float32),
            pltpu.VMEM((BPW,), jnp.float32),
            pltpu.VMEM((BPW,), jnp.float32),
            pltpu.VMEM((BPW,), jnp.float32),
            pltpu.VMEM((16,), jnp.float32),
            pltpu.VMEM((256,), jnp.float32),
            pltpu.SemaphoreType.DMA,
        ],
    )(_body)
    return run(uid, iid, W, U, bias_user, bias_item, gm16)


# speedup vs baseline: 1.5499x; 1.0179x over previous
import functools

import jax
import jax.numpy as jnp
from jax import lax
from jax.experimental import pallas as pl
from jax.experimental.pallas import tpu as pltpu
from jax.experimental.pallas import tpu_sc as plsc

B = 16384
K = 64
NW = 32
BPW = B // NW
CHUNK = 32
NCHUNK = BPW // CHUNK
NITER = NCHUNK // 2
NSEM = 8


def _body(uid_hbm, iid_hbm, w_hbm, u_hbm, bu_hbm, bi_hbm, gm_hbm, out_hbm,
          uid_v, iid_v, wr, ur, bu_v, bi_v, out_v, gm_v, hbuf, bsem, *sems):
    wid = lax.axis_index("s") * 2 + lax.axis_index("c")
    base = wid * BPW

    pltpu.sync_copy(uid_hbm.at[pl.ds(base, BPW)], uid_v)
    pltpu.sync_copy(iid_hbm.at[pl.ds(base, BPW)], iid_v)
    pltpu.sync_copy(gm_hbm, gm_v)

    cbu = pltpu.async_copy(bu_hbm.at[uid_v], bu_v, bsem)
    cbi = pltpu.async_copy(bi_hbm.at[iid_v], bi_v, bsem)

    gm = gm_v[...]
    lanes16 = lax.iota(jnp.int32, 16) * 16

    def fire(c, buf):
        r0 = c * CHUNK
        for h in range(CHUNK // 16):
            uvec = uid_v[pl.ds(r0 + h * 16, 16)]
            ivec = iid_v[pl.ds(r0 + h * 16, 16)]
            for j in range(16):
                row = h * 16 + j
                pltpu.async_copy(
                    w_hbm.at[pl.ds(uvec[j], 1), :],
                    wr.at[buf, pl.ds(row, 1), :], sems[(2 * row) % NSEM])
                pltpu.async_copy(
                    u_hbm.at[pl.ds(ivec[j], 1), :],
                    ur.at[buf, pl.ds(row, 1), :], sems[(2 * row + 1) % NSEM])

    def drain(buf):
        rows_per_sem = 2 * CHUNK // NSEM
        for s in range(NSEM):
            pltpu.make_async_copy(
                w_hbm.at[pl.ds(0, rows_per_sem), :],
                wr.at[buf, pl.ds(0, rows_per_sem), :], sems[s]).wait()

    def compute(c, buf):
        r0 = c * CHUNK
        for g in range(CHUNK // 16):
            for j in range(16):
                row = g * 16 + j
                acc = (wr[buf, row, pl.ds(0, 16)]
                       * ur[buf, row, pl.ds(0, 16)])
                for q in range(1, 4):
                    acc = acc + (wr[buf, row, pl.ds(q * 16, 16)]
                                 * ur[buf, row, pl.ds(q * 16, 16)])
                hbuf[pl.ds(j * 16, 16)] = acc
            o = plsc.load_gather(hbuf, [lanes16])
            for l in range(1, 16):
                o = o + plsc.load_gather(hbuf, [lanes16 + l])
            og = r0 + g * 16
            out_v[pl.ds(og, 16)] = (o + bu_v[pl.ds(og, 16)]
                                    + bi_v[pl.ds(og, 16)] + gm)

    fire(0, 0)

    def body(t, carry):
        fire(2 * t + 1, 1)
        drain(0)
        compute(2 * t, 0)

        @pl.when(t < NITER - 1)
        def _():
            fire(2 * t + 2, 0)

        drain(1)
        compute(2 * t + 1, 1)
        return carry

    lax.fori_loop(0, NITER, body, 0)
    cbu.wait()
    cbi.wait()
    pltpu.sync_copy(out_v, out_hbm.at[pl.ds(base, BPW)])


def kernel(user_ids, item_ids, W, U, bias_user, bias_item, global_mean):
    uid = user_ids.astype(jnp.int32)
    iid = item_ids.astype(jnp.int32)
    gm16 = jnp.broadcast_to(global_mean.astype(jnp.float32), (16,))

    run = functools.partial(
        pl.kernel,
        mesh=plsc.VectorSubcoreMesh(core_axis_name="c", subcore_axis_name="s"),
        compiler_params=pltpu.CompilerParams(needs_layout_passes=False),
        out_type=jax.ShapeDtypeStruct((B,), jnp.float32),
        scratch_types=[
            pltpu.VMEM((BPW,), jnp.int32),
            pltpu.VMEM((BPW,), jnp.int32),
            pltpu.VMEM((2, CHUNK, K), jnp.float32),
            pltpu.VMEM((2, CHUNK, K), jnp.float32),
            pltpu.VMEM((BPW,), jnp.float32),
            pltpu.VMEM((BPW,), jnp.float32),
            pltpu.VMEM((BPW,), jnp.float32),
            pltpu.VMEM((16,), jnp.float32),
            pltpu.VMEM((256,), jnp.float32),
            pltpu.SemaphoreType.DMA,
        ] + [pltpu.SemaphoreType.DMA] * NSEM,
    )(_body)
    return run(uid, iid, W, U, bias_user, bias_item, gm16)
